# baseline (speedup 1.0000x reference)
import functools

import jax
import jax.numpy as jnp
from jax import lax
from jax.experimental import pallas as pl
from jax.experimental.pallas import tpu as pltpu
from jax.experimental.pallas import tpu_sc as plsc

_NC = 2
_NS = 16
_NW = _NC * _NS

_D = 64
_B = 16384 * 50
_B_PER_W = _B // _NW
_CHUNK = 800
_NCHUNK = _B_PER_W // _CHUNK
_NBUF = 2

assert _NCHUNK % _NBUF == 0


def _make_gather():
    mesh = plsc.VectorSubcoreMesh(core_axis_name="c", subcore_axis_name="s")

    @functools.partial(
        pl.kernel,
        out_type=jax.ShapeDtypeStruct((_B, _D), jnp.float32),
        mesh=mesh,
        scratch_types=[
            pltpu.VMEM((_NBUF, _CHUNK), jnp.int32),
            pltpu.VMEM((_NBUF, _CHUNK, _D), jnp.float32),
            [pltpu.SemaphoreType.DMA] * _NBUF,
            [pltpu.SemaphoreType.DMA] * _NBUF,
            [pltpu.SemaphoreType.DMA] * _NBUF,
        ],
        compiler_params=pltpu.CompilerParams(use_tc_tiling_on_sc=False),
    )
    def gather_kernel(idx_hbm, table_hbm, out_hbm, idx_v, rows_v,
                      sem_i, sem_g, sem_o):
        wid = lax.axis_index("s") * _NC + lax.axis_index("c")
        base = wid * _B_PER_W

        def idx_copy(g, b):
            return pltpu.make_async_copy(
                idx_hbm.at[pl.ds(base + g * _CHUNK, _CHUNK)],
                idx_v.at[b], sem_i[b])

        def gather_copy(b):
            return pltpu.make_async_copy(
                table_hbm.at[idx_v.at[b]], rows_v.at[b], sem_g[b])

        def out_copy(g, b):
            return pltpu.make_async_copy(
                rows_v.at[b], out_hbm.at[pl.ds(base + g * _CHUNK, _CHUNK)],
                sem_o[b])

        for b in range(_NBUF):
            idx_copy(b, b).start()

        def ring(r, carry):
            g0 = r * _NBUF
            for b in range(_NBUF):
                g = g0 + b

                @pl.when(r > 0)
                def _(b=b, g=g):
                    out_copy(g - _NBUF, b).wait()

                idx_copy(g, b).wait()
                gather_copy(b).start()

            for b in range(_NBUF):
                g = g0 + b
                gather_copy(b).wait()
                out_copy(g, b).start()

                @pl.when(g + _NBUF < _NCHUNK)
                def _(b=b, g=g):
                    idx_copy(g + _NBUF, b).start()

            return carry

        lax.fori_loop(0, _NCHUNK // _NBUF, ring, 0)

        for b in range(_NBUF):
            out_copy(_NCHUNK - _NBUF + b, b).wait()

    return gather_kernel


_gather = _make_gather()


@jax.jit
def kernel(token_ids, weight):
    idx = token_ids.reshape(-1).astype(jnp.int32)
    out = _gather(idx, weight)
    return out.reshape(token_ids.shape + (weight.shape[1],))


# speedup vs baseline: 1.0060x; 1.0060x over previous
import functools

import jax
import jax.numpy as jnp
from jax import lax
from jax.experimental import pallas as pl
from jax.experimental.pallas import tpu as pltpu
from jax.experimental.pallas import tpu_sc as plsc

_NC = 2
_NS = 16
_NW = _NC * _NS

_D = 64
_B = 16384 * 50
_B_PER_W = _B // _NW
_CHUNK = 200
_NCHUNK = _B_PER_W // _CHUNK
_NBUF = 8

assert _NCHUNK % _NBUF == 0


def _make_gather():
    mesh = plsc.VectorSubcoreMesh(core_axis_name="c", subcore_axis_name="s")

    @functools.partial(
        pl.kernel,
        out_type=jax.ShapeDtypeStruct((_B, _D), jnp.float32),
        mesh=mesh,
        scratch_types=[
            pltpu.VMEM((_NBUF, _CHUNK), jnp.int32),
            pltpu.VMEM((_NBUF, _CHUNK, _D), jnp.float32),
            [pltpu.SemaphoreType.DMA] * _NBUF,
            [pltpu.SemaphoreType.DMA] * _NBUF,
            [pltpu.SemaphoreType.DMA] * _NBUF,
        ],
        compiler_params=pltpu.CompilerParams(use_tc_tiling_on_sc=False),
    )
    def gather_kernel(idx_hbm, table_hbm, out_hbm, idx_v, rows_v,
                      sem_i, sem_g, sem_o):
        wid = lax.axis_index("s") * _NC + lax.axis_index("c")
        base = wid * _B_PER_W

        def idx_copy(g, b):
            return pltpu.make_async_copy(
                idx_hbm.at[pl.ds(base + g * _CHUNK, _CHUNK)],
                idx_v.at[b], sem_i[b])

        def gather_copy(b):
            return pltpu.make_async_copy(
                table_hbm.at[idx_v.at[b]], rows_v.at[b], sem_g[b])

        def out_copy(g, b):
            return pltpu.make_async_copy(
                rows_v.at[b], out_hbm.at[pl.ds(base + g * _CHUNK, _CHUNK)],
                sem_o[b])

        for b in range(_NBUF):
            idx_copy(b, b).start()

        def ring(r, carry):
            g0 = r * _NBUF
            for b in range(_NBUF):
                g = g0 + b

                @pl.when(r > 0)
                def _(b=b, g=g):
                    out_copy(g - _NBUF, b).wait()

                idx_copy(g, b).wait()
                gather_copy(b).start()

            for b in range(_NBUF):
                g = g0 + b
                gather_copy(b).wait()
                out_copy(g, b).start()

                @pl.when(g + _NBUF < _NCHUNK)
                def _(b=b, g=g):
                    idx_copy(g + _NBUF, b).start()

            return carry

        lax.fori_loop(0, _NCHUNK // _NBUF, ring, 0)

        for b in range(_NBUF):
            out_copy(_NCHUNK - _NBUF + b, b).wait()

    return gather_kernel


_gather = _make_gather()


@jax.jit
def kernel(token_ids, weight):
    idx = token_ids.reshape(-1).astype(jnp.int32)
    out = _gather(idx, weight)
    return out.reshape(token_ids.shape + (weight.shape[1],))
